# baseline (speedup 1.0000x reference)
import jax
import jax.numpy as jnp
from jax.experimental import pallas as pl
from jax.experimental.pallas import tpu as pltpu

_DEPTH = 6
_UNROLL = 2


def _pool_body(x_hbm, c_ref, w_ref, b_ref, o_ref, bufs, sems, asum_s):
    B = x_hbm.shape[0]
    w = w_ref[...]
    bvec = b_ref[...]
    K = w.shape[0]

    def dma_in(slot, b):
        pltpu.make_async_copy(x_hbm.at[b], bufs.at[slot], sems.at[slot]).start()

    for i in range(_DEPTH):
        dma_in(i, i)

    def compute_one(slot, b):
        x = bufs[slot]
        logits = jax.lax.dot_general(
            w, x, (((1,), (1,)), ((), ())), preferred_element_type=jnp.float32
        )
        logits = logits + bvec
        m = jnp.max(logits, axis=0, keepdims=True)
        e = jnp.exp(logits - m)
        s = jnp.sum(e, axis=0, keepdims=True)
        a = e / s
        ax = jax.lax.dot_general(
            a, x, (((1,), (0,)), ((), ())), preferred_element_type=jnp.float32
        )
        asum = jnp.sum(a, axis=1, keepdims=True)
        o_ref[b] = ax
        asum_s[b] = jnp.broadcast_to(asum, (K, 128))

    def body(it, _):
        b0 = it * _UNROLL
        slots = [jax.lax.rem(b0 + u, _DEPTH) for u in range(_UNROLL)]
        for u in range(_UNROLL):
            pltpu.make_async_copy(
                bufs.at[slots[u]], bufs.at[slots[u]], sems.at[slots[u]]
            ).wait()
        for u in range(_UNROLL):
            compute_one(slots[u], b0 + u)
        for u in range(_UNROLL):
            @pl.when(b0 + u + _DEPTH < B)
            def _(u=u):
                dma_in(slots[u], b0 + u + _DEPTH)
        return ()

    jax.lax.fori_loop(0, B // _UNROLL, body, ())

    ax_all = o_ref[...]
    asum_all = asum_s[...][:, :, 0:1]
    pooled = ax_all - asum_all * c_ref[...][None]
    ss = jnp.sum(pooled * pooled, axis=2, keepdims=True)
    ss = jnp.sum(ss, axis=1, keepdims=True)
    norm = jnp.maximum(jnp.sqrt(ss), 1e-12)
    o_ref[...] = pooled / norm


def kernel(x, centers, attn_w, attn_b):
    B, T, D = x.shape
    K = centers.shape[0]
    out = pl.pallas_call(
        _pool_body,
        out_shape=jax.ShapeDtypeStruct((B, K, D), x.dtype),
        in_specs=[
            pl.BlockSpec(memory_space=pl.ANY),
            pl.BlockSpec((K, D), lambda: (0, 0)),
            pl.BlockSpec((K, D), lambda: (0, 0)),
            pl.BlockSpec((K, 1), lambda: (0, 0)),
        ],
        out_specs=pl.BlockSpec((B, K, D), lambda: (0, 0, 0)),
        scratch_shapes=[
            pltpu.VMEM((_DEPTH, T, D), jnp.float32),
            pltpu.SemaphoreType.DMA((_DEPTH,)),
            pltpu.VMEM((B, K, 128), jnp.float32),
        ],
        compiler_params=pltpu.CompilerParams(
            vmem_limit_bytes=48 * 1024 * 1024,
        ),
        name="temporal_pooling",
    )(x, centers, attn_w, attn_b.reshape(K, 1))
    return out.reshape(B, K * D)


# speedup vs baseline: 1.0341x; 1.0341x over previous
import jax
import jax.numpy as jnp
from jax.experimental import pallas as pl
from jax.experimental.pallas import tpu as pltpu

_DEPTH = 3


def _pool_body(x_hbm, c_ref, w_ref, b_ref, o_ref, bufs, sems):
    B = x_hbm.shape[0]
    w = w_ref[...]
    c = c_ref[...]
    bvec = b_ref[...]

    def dma_in(slot, b):
        pltpu.make_async_copy(x_hbm.at[b], bufs.at[slot], sems.at[slot]).start()

    for i in range(_DEPTH):
        dma_in(i, i)

    def body(b, _):
        slot = jax.lax.rem(b, _DEPTH)
        pltpu.make_async_copy(bufs.at[slot], bufs.at[slot], sems.at[slot]).wait()
        x = bufs[slot]
        logits = jax.lax.dot_general(
            w, x, (((1,), (1,)), ((), ())), preferred_element_type=jnp.float32
        )
        logits = logits + bvec
        m = jnp.max(logits, axis=0, keepdims=True)
        e = jnp.exp(logits - m)
        s = jnp.sum(e, axis=0, keepdims=True)
        a = e / s
        ax = jax.lax.dot_general(
            a, x, (((1,), (0,)), ((), ())), preferred_element_type=jnp.float32
        )
        asum = jnp.sum(a, axis=1, keepdims=True)
        pooled = ax - asum * c
        ss = jnp.sum(pooled * pooled, axis=1, keepdims=True)
        ss = jnp.sum(ss, axis=0, keepdims=True)
        norm = jnp.maximum(jnp.sqrt(ss), 1e-12)
        o_ref[b] = pooled / norm

        @pl.when(b + _DEPTH < B)
        def _():
            dma_in(slot, b + _DEPTH)

        return ()

    jax.lax.fori_loop(0, B, body, ())


def kernel(x, centers, attn_w, attn_b):
    B, T, D = x.shape
    K = centers.shape[0]
    out = pl.pallas_call(
        _pool_body,
        out_shape=jax.ShapeDtypeStruct((B, K, D), x.dtype),
        in_specs=[
            pl.BlockSpec(memory_space=pl.ANY),
            pl.BlockSpec((K, D), lambda: (0, 0)),
            pl.BlockSpec((K, D), lambda: (0, 0)),
            pl.BlockSpec((K, 1), lambda: (0, 0)),
        ],
        out_specs=pl.BlockSpec((B, K, D), lambda: (0, 0, 0)),
        scratch_shapes=[
            pltpu.VMEM((_DEPTH, T, D), jnp.float32),
            pltpu.SemaphoreType.DMA((_DEPTH,)),
        ],
        compiler_params=pltpu.CompilerParams(
            vmem_limit_bytes=48 * 1024 * 1024,
        ),
        name="temporal_pooling",
    )(x, centers, attn_w, attn_b.reshape(K, 1))
    return out.reshape(B, K * D)
